# baseline (speedup 1.0000x reference)
import functools

import jax
import jax.numpy as jnp
from jax import lax
from jax.experimental import pallas as pl
from jax.experimental.pallas import tpu as pltpu
from jax.experimental.pallas import tpu_sc as plsc

_D = 50
_DP = 56
_L = 16
_NW = 32
_CHUNK = 256
_KSUB = _CHUNK // 128
_GROUPS = _CHUNK // _L
_UNROLL = 6


def _transe_body(n_chunks, n_iters, h_hbm, g_hbm, ei_hbm, et_hbm, out_hbm,
                 g_loc,
                 irc0, it0, irc1, it1, irc2, it2,
                 rows0, cols0, out0, rows1, cols1, out1,
                 sem_g, sem_i0, sem_i1, sem_i2,
                 sem_r0, sem_c0, sem_r1, sem_c1, sem_o0, sem_o1):
    cid = lax.axis_index("c")
    sid = lax.axis_index("s")
    wid = sid * 2 + cid

    pltpu.async_copy(g_hbm, g_loc, sem_g).wait()

    idxb = ((irc0, it0, sem_i0), (irc1, it1, sem_i1), (irc2, it2, sem_i2))
    datb = ((rows0, cols0, out0, sem_r0, sem_c0, sem_o0),
            (rows1, cols1, out1, sem_r1, sem_c1, sem_o1))

    def valid(i):
        return (i < n_iters) & ((i * _NW + wid) < n_chunks)

    def idx_copy(i, s3):
        irc, it, si = idxb[s3]

        @pl.when(valid(i))
        def _():
            base = (i * _NW + wid) * _CHUNK
            pltpu.async_copy(ei_hbm.at[:, pl.ds(base, _CHUNK)], irc, si)
            pltpu.async_copy(et_hbm.at[pl.ds(base, _CHUNK)], it, si)

    def idx_wait(i, s3):
        irc, it, si = idxb[s3]

        @pl.when(valid(i))
        def _():
            base = (i * _NW + wid) * _CHUNK
            pltpu.make_async_copy(
                ei_hbm.at[:, pl.ds(base, _CHUNK)], irc, si).wait()
            pltpu.make_async_copy(et_hbm.at[pl.ds(base, _CHUNK)], it, si).wait()

    def fire_gathers(i, s3, s2):
        irc, it, si = idxb[s3]
        rows_v, cols_v, _, sr, sc, _ = datb[s2]

        @pl.when(valid(i))
        def _():
            for k in range(_KSUB):
                pltpu.async_copy(
                    h_hbm.at[irc.at[0, pl.ds(k * 128, 128)]],
                    rows_v.at[pl.ds(k * 128, 128)], sr)
                pltpu.async_copy(
                    h_hbm.at[irc.at[1, pl.ds(k * 128, 128)]],
                    cols_v.at[pl.ds(k * 128, 128)], sc)

    def compute(i, s3, s2, out_wait):
        irc, it, si = idxb[s3]
        rows_v, cols_v, out_v, sr, sc, so = datb[s2]

        @pl.when(valid(i))
        def _():
            base = (i * _NW + wid) * _CHUNK
            for k in range(_KSUB):
                pltpu.make_async_copy(
                    h_hbm.at[irc.at[0, pl.ds(k * 128, 128)]],
                    rows_v.at[pl.ds(k * 128, 128)], sr).wait()
                pltpu.make_async_copy(
                    h_hbm.at[irc.at[1, pl.ds(k * 128, 128)]],
                    cols_v.at[pl.ds(k * 128, 128)], sc).wait()

            @pl.when(out_wait)
            def _():
                pltpu.make_async_copy(
                    out_v, out_hbm.at[pl.ds(base, _CHUNK)], so).wait()

            @plsc.parallel_loop(0, _GROUPS)
            def group_body(g16):
                ebase = g16 * _L
                lane = lax.iota(jnp.int32, _L)
                eidx = ebase + lane
                typ = it[pl.ds(ebase, _L)]
                acc = jnp.zeros((_L,), jnp.float32)
                for d in range(_D):
                    dsplat = jnp.full((_L,), d, jnp.int32)
                    r = plsc.load_gather(rows_v, [eidx, dsplat])
                    cl = plsc.load_gather(cols_v, [eidx, dsplat])
                    rl = plsc.load_gather(g_loc, [typ, dsplat])
                    acc = acc + jnp.abs(r + rl - cl)
                out_v[pl.ds(ebase, _L)] = acc
            pltpu.async_copy(out_v, out_hbm.at[pl.ds(base, _CHUNK)], so)

    idx_copy(0, 0)
    idx_copy(1, 1)
    idx_wait(0, 0)
    fire_gathers(0, 0, 0)

    def six_body(m, carry):
        i0 = m * _UNROLL
        for j in range(_UNROLL):
            i = i0 + j
            idx_copy(i + 2, (j + 2) % 3)
            idx_wait(i + 1, (j + 1) % 3)
            fire_gathers(i + 1, (j + 1) % 3, (j + 1) % 2)
            compute(i, j % 3, j % 2, out_wait=(i >= 2))
        return carry

    n_six = (n_iters + _UNROLL - 1) // _UNROLL
    lax.fori_loop(0, n_six, six_body, 0)

    def drain(i):
        rows_v, cols_v, out_v, sr, sc, so = datb[i % 2]

        @pl.when(valid(i) & ~valid(i + 2))
        def _():
            base = (i * _NW + wid) * _CHUNK
            pltpu.make_async_copy(
                out_v, out_hbm.at[pl.ds(base, _CHUNK)], so).wait()

    for i in (n_iters - 3, n_iters - 2, n_iters - 1):
        drain(i)


def kernel(h, g, edge_idx, edge_type):
    n, d = h.shape
    r_rel, _ = g.shape
    e = edge_type.shape[0]
    assert d == _D
    n_chunks = e // _CHUNK
    assert n_chunks * _CHUNK == e
    n_iters = (n_chunks + _NW - 1) // _NW

    ei = edge_idx.astype(jnp.int32)
    et = edge_type.astype(jnp.int32)
    hp = jnp.pad(h.astype(jnp.float32), ((0, 0), (0, _DP - _D)))

    mesh = plsc.VectorSubcoreMesh(core_axis_name="c", subcore_axis_name="s")
    body = functools.partial(_transe_body, n_chunks, n_iters)
    idxset = lambda: [
        pltpu.VMEM((2, _CHUNK), jnp.int32),
        pltpu.VMEM((_CHUNK,), jnp.int32),
    ]
    datset = lambda: [
        pltpu.VMEM((_CHUNK, _DP), jnp.float32),
        pltpu.VMEM((_CHUNK, _DP), jnp.float32),
        pltpu.VMEM((_CHUNK,), jnp.float32),
    ]
    run = pl.kernel(
        body,
        out_type=jax.ShapeDtypeStruct((e,), jnp.float32),
        mesh=mesh,
        compiler_params=pltpu.CompilerParams(
            needs_layout_passes=False, use_tc_tiling_on_sc=False
        ),
        scratch_types=(
            [pltpu.VMEM((r_rel, _D), jnp.float32)]
            + idxset() + idxset() + idxset()
            + datset() + datset()
            + [pltpu.SemaphoreType.DMA] * 10
        ),
    )
    return run(hp, g.astype(jnp.float32), ei, et)


# speedup vs baseline: 1.6470x; 1.6470x over previous
import functools

import jax
import jax.numpy as jnp
from jax import lax
from jax.experimental import pallas as pl
from jax.experimental.pallas import tpu as pltpu
from jax.experimental.pallas import tpu_sc as plsc

_D = 50
_DP = 56
_L = 16
_NW = 32
_CHUNK = 256
_KSUB = _CHUNK // 128
_GROUPS = _CHUNK // _L
_UNROLL = 6


def _transe_body(n_chunks, n_iters, h_hbm, g_hbm, ei_hbm, et_hbm, out_hbm,
                 g_loc,
                 irc0, it0, irc1, it1, irc2, it2,
                 rows0, cols0, out0, rows1, cols1, out1,
                 sem_g, sem_i0, sem_i1, sem_i2,
                 sem_r0, sem_c0, sem_r1, sem_c1, sem_o0, sem_o1):
    cid = lax.axis_index("c")
    sid = lax.axis_index("s")
    wid = sid * 2 + cid

    pltpu.async_copy(g_hbm, g_loc, sem_g).wait()

    idxb = ((irc0, it0, sem_i0), (irc1, it1, sem_i1), (irc2, it2, sem_i2))
    datb = ((rows0, cols0, out0, sem_r0, sem_c0, sem_o0),
            (rows1, cols1, out1, sem_r1, sem_c1, sem_o1))

    def valid(i):
        return (i < n_iters) & ((i * _NW + wid) < n_chunks)

    def idx_copy(i, s3):
        irc, it, si = idxb[s3]

        @pl.when(valid(i))
        def _():
            base = (i * _NW + wid) * _CHUNK
            pltpu.async_copy(ei_hbm.at[:, pl.ds(base, _CHUNK)], irc, si)
            pltpu.async_copy(et_hbm.at[pl.ds(base, _CHUNK)], it, si)

    def idx_wait(i, s3):
        irc, it, si = idxb[s3]

        @pl.when(valid(i))
        def _():
            base = (i * _NW + wid) * _CHUNK
            pltpu.make_async_copy(
                ei_hbm.at[:, pl.ds(base, _CHUNK)], irc, si).wait()
            pltpu.make_async_copy(et_hbm.at[pl.ds(base, _CHUNK)], it, si).wait()

    def fire_gathers(i, s3, s2):
        irc, it, si = idxb[s3]
        rows_v, cols_v, _, sr, sc, _ = datb[s2]

        @pl.when(valid(i))
        def _():
            for k in range(_KSUB):
                pltpu.async_copy(
                    h_hbm.at[irc.at[0, pl.ds(k * 128, 128)]],
                    rows_v.at[pl.ds(k * 128, 128)], sr)
                pltpu.async_copy(
                    h_hbm.at[irc.at[1, pl.ds(k * 128, 128)]],
                    cols_v.at[pl.ds(k * 128, 128)], sc)

    def compute(i, s3, s2, out_wait):
        irc, it, si = idxb[s3]
        rows_v, cols_v, out_v, sr, sc, so = datb[s2]

        @pl.when(valid(i))
        def _():
            base = (i * _NW + wid) * _CHUNK
            for k in range(_KSUB):
                pltpu.make_async_copy(
                    h_hbm.at[irc.at[0, pl.ds(k * 128, 128)]],
                    rows_v.at[pl.ds(k * 128, 128)], sr).wait()
                pltpu.make_async_copy(
                    h_hbm.at[irc.at[1, pl.ds(k * 128, 128)]],
                    cols_v.at[pl.ds(k * 128, 128)], sc).wait()

            @pl.when(out_wait)
            def _():
                pltpu.make_async_copy(
                    out_v, out_hbm.at[pl.ds(base, _CHUNK)], so).wait()

            def group_body(g16, carry2):
                ebase = g16 * _L
                lane = lax.iota(jnp.int32, _L)
                eidx = ebase + lane
                typ = it[pl.ds(ebase, _L)]
                acc = jnp.zeros((_L,), jnp.float32)
                for d in range(_D):
                    dsplat = jnp.full((_L,), d, jnp.int32)
                    r = plsc.load_gather(rows_v, [eidx, dsplat])
                    cl = plsc.load_gather(cols_v, [eidx, dsplat])
                    rl = plsc.load_gather(g_loc, [typ, dsplat])
                    acc = acc + jnp.abs(r + rl - cl)
                out_v[pl.ds(ebase, _L)] = acc
                return carry2

            lax.fori_loop(0, _GROUPS, group_body, 0)
            pltpu.async_copy(out_v, out_hbm.at[pl.ds(base, _CHUNK)], so)

    idx_copy(0, 0)
    idx_copy(1, 1)
    idx_wait(0, 0)
    fire_gathers(0, 0, 0)

    def six_body(m, carry):
        i0 = m * _UNROLL
        for j in range(_UNROLL):
            i = i0 + j
            idx_copy(i + 2, (j + 2) % 3)
            idx_wait(i + 1, (j + 1) % 3)
            fire_gathers(i + 1, (j + 1) % 3, (j + 1) % 2)
            compute(i, j % 3, j % 2, out_wait=(i >= 2))
        return carry

    n_six = (n_iters + _UNROLL - 1) // _UNROLL
    lax.fori_loop(0, n_six, six_body, 0)

    def drain(i):
        rows_v, cols_v, out_v, sr, sc, so = datb[i % 2]

        @pl.when(valid(i) & ~valid(i + 2))
        def _():
            base = (i * _NW + wid) * _CHUNK
            pltpu.make_async_copy(
                out_v, out_hbm.at[pl.ds(base, _CHUNK)], so).wait()

    for i in (n_iters - 3, n_iters - 2, n_iters - 1):
        drain(i)


def kernel(h, g, edge_idx, edge_type):
    n, d = h.shape
    r_rel, _ = g.shape
    e = edge_type.shape[0]
    assert d == _D
    n_chunks = e // _CHUNK
    assert n_chunks * _CHUNK == e
    n_iters = (n_chunks + _NW - 1) // _NW

    ei = edge_idx.astype(jnp.int32)
    et = edge_type.astype(jnp.int32)
    hp = jnp.pad(h.astype(jnp.float32), ((0, 0), (0, _DP - _D)))

    mesh = plsc.VectorSubcoreMesh(core_axis_name="c", subcore_axis_name="s")
    body = functools.partial(_transe_body, n_chunks, n_iters)
    idxset = lambda: [
        pltpu.VMEM((2, _CHUNK), jnp.int32),
        pltpu.VMEM((_CHUNK,), jnp.int32),
    ]
    datset = lambda: [
        pltpu.VMEM((_CHUNK, _DP), jnp.float32),
        pltpu.VMEM((_CHUNK, _DP), jnp.float32),
        pltpu.VMEM((_CHUNK,), jnp.float32),
    ]
    run = pl.kernel(
        body,
        out_type=jax.ShapeDtypeStruct((e,), jnp.float32),
        mesh=mesh,
        compiler_params=pltpu.CompilerParams(
            needs_layout_passes=False, use_tc_tiling_on_sc=False
        ),
        scratch_types=(
            [pltpu.VMEM((r_rel, _D), jnp.float32)]
            + idxset() + idxset() + idxset()
            + datset() + datset()
            + [pltpu.SemaphoreType.DMA] * 10
        ),
    )
    return run(hp, g.astype(jnp.float32), ei, et)
